# Initial kernel scaffold; baseline (speedup 1.0000x reference)
import jax
import jax.numpy as jnp
from jax.experimental import pallas as pl


def kernel(x, img_size, W1, b1, Ws, bs, Wl, bl):
    raise NotImplementedError("write your pallas kernel here")


# speedup vs baseline: 155.2223x; 155.2223x over previous
import functools

import jax
import jax.numpy as jnp
import numpy as np
from jax.experimental import pallas as pl

RATIOS = [0.5, 1.0, 2.0]
SCALES = [8, 16, 32]
FEAT_STRIDE = 16
NMS_THRESH = 0.7
N_PRE_NMS = 12000
N_POST_NMS = 600
MIN_SIZE = 16

TILE = 512
NTILES = 24

_pt = np.zeros((N_POST_NMS, N_POST_NMS), dtype=np.int32)
for _m in range(1, N_POST_NMS):
    _rng = np.random.RandomState(0)
    _extra = _rng.choice(_m, size=N_POST_NMS - _m, replace=True)
    _pt[_m] = np.concatenate([np.arange(_m), _extra]).astype(np.int32)
_PAD_TABLE = jnp.asarray(_pt)


def _anchor_base(base_size=16):
    ab = np.zeros((len(RATIOS) * len(SCALES), 4), dtype=np.float32)
    for i, r in enumerate(RATIOS):
        for j, s in enumerate(SCALES):
            h = base_size * s * np.sqrt(r)
            w = base_size * s * np.sqrt(1.0 / r)
            k = i * len(SCALES) + j
            ab[k] = [-w / 2.0, -h / 2.0, w / 2.0, h / 2.0]
    return ab


def _shifted_anchors(stride, h, w):
    ab = _anchor_base()
    sx = np.arange(0, w * stride, stride)
    sy = np.arange(0, h * stride, stride)
    mx, my = np.meshgrid(sx, sy)
    shift = np.stack([mx.ravel(), my.ravel(), mx.ravel(), my.ravel()], axis=1).astype(np.float32)
    anc = (ab[None, :, :] + shift[:, None, :]).reshape(-1, 4)
    return anc.astype(np.float32)


def _nms_kernel(x1_ref, y1_ref, x2_ref, y2_ref, act_ref, keep_ref):
    def area_row(i):
        x1 = x1_ref[i, :]
        y1 = y1_ref[i, :]
        x2 = x2_ref[i, :]
        y2 = y2_ref[i, :]
        return x1, y1, x2, y2, (x2 - x1) * (y2 - y1)

    def pair_overlap(jbox, ibox):
        (jx1, jy1, jx2, jy2, ja) = jbox
        (ix1, iy1, ix2, iy2, ia) = ibox
        jx1 = jx1.reshape(TILE, 1)
        jy1 = jy1.reshape(TILE, 1)
        jx2 = jx2.reshape(TILE, 1)
        jy2 = jy2.reshape(TILE, 1)
        ja = ja.reshape(TILE, 1)
        ix1 = ix1.reshape(1, TILE)
        iy1 = iy1.reshape(1, TILE)
        ix2 = ix2.reshape(1, TILE)
        iy2 = iy2.reshape(1, TILE)
        ia = ia.reshape(1, TILE)
        xx1 = jnp.maximum(jx1, ix1)
        yy1 = jnp.maximum(jy1, iy1)
        xx2 = jnp.minimum(jx2, ix2)
        yy2 = jnp.minimum(jy2, iy2)
        inter = jnp.maximum(0.0, xx2 - xx1) * jnp.maximum(0.0, yy2 - yy1)
        iou = inter / (ja + ia - inter + 1e-12)
        return iou > NMS_THRESH

    def outer(i, _):
        ibox = area_row(i)

        def cross(j, sup):
            jbox = area_row(j)
            ov = pair_overlap(jbox, ibox)
            keep_j = keep_ref[j, :].reshape(TILE, 1)
            hit = jnp.where(ov, keep_j, 0.0)
            return jnp.maximum(sup, jnp.max(hit, axis=0, keepdims=True))

        sup = jax.lax.fori_loop(0, i, cross, jnp.zeros((1, TILE), jnp.float32))
        act = act_ref[i, :].reshape(1, TILE) * (1.0 - sup)

        ov_ii = pair_overlap(ibox, ibox)
        row_lt_col = jax.lax.broadcasted_iota(jnp.int32, (TILE, TILE), 0) < \
            jax.lax.broadcasted_iota(jnp.int32, (TILE, TILE), 1)
        o_mat = jnp.where(ov_ii & row_lt_col, 1.0, 0.0)

        def w_cond(carry):
            _, changed = carry
            return changed

        def w_body(carry):
            keep_cur, _ = carry
            s = jnp.max(o_mat * keep_cur.reshape(TILE, 1), axis=0, keepdims=True)
            new = jnp.where(s > 0.0, 0.0, act)
            changed = jnp.any(new != keep_cur)
            return new, changed

        keep_i, _ = jax.lax.while_loop(w_cond, w_body, (act, jnp.bool_(True)))
        keep_ref[i, :] = keep_i.reshape(TILE)
        return 0

    jax.lax.fori_loop(0, NTILES, outer, 0, unroll=False)


@functools.partial(jax.jit, static_argnames=())
def _run_nms(x1, y1, x2, y2, act):
    return pl.pallas_call(
        _nms_kernel,
        out_shape=jax.ShapeDtypeStruct((NTILES, TILE), jnp.float32),
    )(x1, y1, x2, y2, act)


def kernel(x, img_size, W1, b1, Ws, bs, Wl, bl):
    n, _, h, w = x.shape
    f = jax.nn.relu(
        jax.lax.conv_general_dilated(
            x, W1, (1, 1), ((1, 1), (1, 1)),
            dimension_numbers=("NCHW", "OIHW", "NCHW")) + b1[None, :, None, None])
    rpn_locs = jnp.transpose(
        jax.lax.conv_general_dilated(
            f, Wl, (1, 1), ((0, 0), (0, 0)),
            dimension_numbers=("NCHW", "OIHW", "NCHW")) + bl[None, :, None, None],
        (0, 2, 3, 1)).reshape(n, -1, 4)
    rpn_scores = jnp.transpose(
        jax.lax.conv_general_dilated(
            f, Ws, (1, 1), ((0, 0), (0, 0)),
            dimension_numbers=("NCHW", "OIHW", "NCHW")) + bs[None, :, None, None],
        (0, 2, 3, 1)).reshape(n, -1, 2)
    fg = jax.nn.softmax(rpn_scores, axis=-1)[:, :, 1].reshape(n, -1)

    anchor = _shifted_anchors(FEAT_STRIDE, h, w)
    anc = jnp.asarray(anchor)
    loc = rpn_locs[0]
    score = fg[0]

    aw = anc[:, 2] - anc[:, 0]
    ah = anc[:, 3] - anc[:, 1]
    acx = anc[:, 0] + 0.5 * aw
    acy = anc[:, 1] + 0.5 * ah
    ncx = loc[:, 0] * aw + acx
    ncy = loc[:, 1] * ah + acy
    nw = jnp.exp(loc[:, 2]) * aw
    nh = jnp.exp(loc[:, 3]) * ah
    iw = img_size[1].astype(jnp.float32)
    ih = img_size[0].astype(jnp.float32)
    rx1 = jnp.clip(ncx - 0.5 * nw, 0.0, iw)
    ry1 = jnp.clip(ncy - 0.5 * nh, 0.0, ih)
    rx2 = jnp.clip(ncx + 0.5 * nw, 0.0, iw)
    ry2 = jnp.clip(ncy + 0.5 * nh, 0.0, ih)

    ms = MIN_SIZE * 1.0
    mask = (rx2 - rx1 >= ms) & (ry2 - ry1 >= ms)
    sort_key = jnp.where(mask, -score, jnp.inf)
    order = jnp.argsort(sort_key, stable=True)[:N_PRE_NMS]
    nvalid = jnp.sum(mask.astype(jnp.int32))

    K = N_PRE_NMS
    pad = NTILES * TILE - K
    x1s = jnp.pad(rx1[order], (0, pad)).reshape(NTILES, TILE)
    y1s = jnp.pad(ry1[order], (0, pad)).reshape(NTILES, TILE)
    x2s = jnp.pad(rx2[order], (0, pad)).reshape(NTILES, TILE)
    y2s = jnp.pad(ry2[order], (0, pad)).reshape(NTILES, TILE)
    act = (jnp.arange(NTILES * TILE) < jnp.minimum(nvalid, K)).astype(
        jnp.float32).reshape(NTILES, TILE)

    keep_f = _run_nms(x1s, y1s, x2s, y2s, act)
    keep = keep_f.reshape(-1)[:K] > 0.0

    M = jnp.sum(keep.astype(jnp.int32))
    pos = jnp.cumsum(keep.astype(jnp.int32)) - 1
    kept_map = jnp.zeros((K,), jnp.int32).at[jnp.where(keep, pos, K)].set(
        jnp.arange(K, dtype=jnp.int32), mode="drop")
    sel = jnp.where(
        M >= N_POST_NMS,
        jnp.arange(N_POST_NMS, dtype=jnp.int32),
        _PAD_TABLE[jnp.clip(M, 0, N_POST_NMS - 1)],
    )
    roi_s = jnp.stack([rx1[order], ry1[order], rx2[order], ry2[order]], axis=1)
    rois = roi_s[kept_map[sel]]

    rois_indices = jnp.zeros((1, N_POST_NMS), jnp.float32)
    anchor_out = anc[None].astype(jnp.float32)
    return (rpn_locs, rpn_scores, rois, rois_indices, anchor_out)
